# Initial kernel scaffold; baseline (speedup 1.0000x reference)
import jax
import jax.numpy as jnp
from jax.experimental import pallas as pl


def kernel(data, matrix, conv_W, conv_b, fc1_W, fc1_b, fc2_W, fc2_b):
    raise NotImplementedError("write your pallas kernel here")


# speedup vs baseline: 4663.4503x; 4663.4503x over previous
import jax
import jax.numpy as jnp
from jax.experimental import pallas as pl
from jax.experimental.pallas import tpu as pltpu

_CONTRACT0 = (((0,), (0,)), ((), ()))


def _net_kernel(data_ref, matrix_ref, conv_W_ref, conv_b_ref,
                fc1_W_ref, fc1_b_ref, fc2_W_ref, fc2_b_ref, out_ref):
    f32 = jnp.float32
    a = matrix_ref[...].astype(f32)
    n = a.shape[0]

    ones_col = jnp.ones((n, 1), dtype=f32)
    deg = jax.lax.dot_general(a, ones_col, _CONTRACT0,
                              preferred_element_type=f32) + 1.0
    dinv = jax.lax.rsqrt(deg)

    xw = jnp.dot(data_ref[...], conv_W_ref[...],
                 preferred_element_type=f32)
    z = xw * dinv
    agg = jax.lax.dot_general(a, z, _CONTRACT0,
                              preferred_element_type=f32) + z
    h = jnp.maximum(agg * dinv + conv_b_ref[...], 0.0)

    h = jnp.maximum(jnp.dot(h, fc1_W_ref[...],
                            preferred_element_type=f32) + fc1_b_ref[...], 0.0)
    out_ref[...] = jnp.dot(h, fc2_W_ref[...],
                           preferred_element_type=f32) + fc2_b_ref[...]


def kernel(data, matrix, conv_W, conv_b, fc1_W, fc1_b, fc2_W, fc2_b):
    n, _ = data.shape
    o = fc2_W.shape[1]
    return pl.pallas_call(
        _net_kernel,
        out_shape=jax.ShapeDtypeStruct((n, o), jnp.float32),
    )(data, matrix, conv_W, conv_b.reshape(1, -1),
      fc1_W, fc1_b.reshape(1, -1), fc2_W, fc2_b.reshape(1, -1))


# speedup vs baseline: 5608.7821x; 1.2027x over previous
import jax
import jax.numpy as jnp
from jax.experimental import pallas as pl
from jax.experimental.pallas import tpu as pltpu

_CONTRACT0 = (((0,), (0,)), ((), ()))


def _net_kernel(data_ref, matrix_ref, conv_W_ref, conv_b_ref,
                fc1_W_ref, fc1_b_ref, fc2_W_ref, fc2_b_ref, out_ref):
    f32, bf16 = jnp.float32, jnp.bfloat16
    a = matrix_ref[...].astype(f32)
    a_bf = a.astype(bf16)

    deg = jnp.sum(a, axis=0, keepdims=True) + 1.0
    dinv = jnp.transpose(jax.lax.rsqrt(deg))

    xw = jnp.dot(data_ref[...], conv_W_ref[...],
                 preferred_element_type=f32)
    z = xw * dinv
    z_hi = z.astype(bf16)
    z_lo = (z - z_hi.astype(f32)).astype(bf16)
    rhs = jnp.concatenate([z_hi, z_lo], axis=1)
    agg2 = jax.lax.dot_general(a_bf, rhs, _CONTRACT0,
                               preferred_element_type=f32)
    h = agg2[:, :z.shape[1]] + agg2[:, z.shape[1]:] + z
    h = jnp.maximum(h * dinv + conv_b_ref[...], 0.0)

    h = jnp.maximum(jnp.dot(h, fc1_W_ref[...],
                            preferred_element_type=f32) + fc1_b_ref[...], 0.0)
    out_ref[...] = jnp.dot(h, fc2_W_ref[...],
                           preferred_element_type=f32) + fc2_b_ref[...]


def kernel(data, matrix, conv_W, conv_b, fc1_W, fc1_b, fc2_W, fc2_b):
    n, _ = data.shape
    o = fc2_W.shape[1]
    return pl.pallas_call(
        _net_kernel,
        out_shape=jax.ShapeDtypeStruct((n, o), jnp.float32),
    )(data, matrix, conv_W, conv_b.reshape(1, -1),
      fc1_W, fc1_b.reshape(1, -1), fc2_W, fc2_b.reshape(1, -1))
